# baseline (speedup 1.0000x reference)
import functools

import jax
import jax.numpy as jnp
from jax import lax
from jax.experimental import pallas as pl
from jax.experimental.pallas import tpu as pltpu
from jax.experimental.pallas import tpu_sc as plsc

N = 16384
L = 16
NT = 16
ROWS = N // NT
CHUNKS = ROWS // L

_mesh = plsc.VectorSubcoreMesh(
    core_axis_name="c", subcore_axis_name="s", num_cores=2)


@functools.partial(
    pl.kernel,
    mesh=_mesh,
    out_type=jax.ShapeDtypeStruct((L,), jnp.float32),
    compiler_params=pltpu.CompilerParams(needs_layout_passes=False),
    scratch_types=[
        pltpu.VMEM((2 * ROWS,), jnp.float32),
        pltpu.VMEM((ROWS,), jnp.int32),
        pltpu.VMEM((3 * L,), jnp.float32),
        pltpu.VMEM((NT * 3 * L,), jnp.float32),
        pltpu.VMEM_SHARED((NT * 3 * L,), jnp.float32),
        pltpu.VMEM((L,), jnp.float32),
        pltpu.SemaphoreType.DMA,
        pltpu.SemaphoreType.DMA,
    ],
)
def _diffdp(y_hbm, s_hbm, out_hbm, y_v, s_v, part_v, stage_v, shared, res_v,
            sem_y, sem_s):
    tid = lax.axis_index("s")
    base = tid * ROWS
    y_cp = pltpu.async_copy(y_hbm.at[pl.ds(2 * base, 2 * ROWS)], y_v, sem_y)
    s_cp = pltpu.async_copy(s_hbm.at[pl.ds(base, ROWS)], s_v, sem_s)
    s_cp.wait()
    y_cp.wait()

    zf = jnp.zeros((L,), jnp.float32)

    @plsc.parallel_loop(0, CHUNKS, carry=(zf, zf, zf), unroll=4)
    def acc(i, carry):
        acc_y, acc_ys, acc_s = carry
        off = L * i + 128 * (i // 8) + 128
        yv = y_v[pl.ds(off, L)]
        sv = s_v[pl.ds(i * L, L)].astype(jnp.float32)
        return acc_y + yv, acc_ys + yv * sv, acc_s + sv

    acc_y, acc_ys, acc_s = acc
    part_v[pl.ds(0, L)] = acc_y
    part_v[pl.ds(L, L)] = acc_ys
    part_v[pl.ds(2 * L, L)] = acc_s
    pltpu.sync_copy(part_v, shared.at[pl.ds(tid * 3 * L, 3 * L)])
    plsc.subcore_barrier()

    @pl.when(tid == 0)
    def _():
        pltpu.sync_copy(shared, stage_v)

        def comb(k, carry):
            a_y, a_ys, a_s = carry
            o = k * 3 * L
            return (a_y + stage_v[pl.ds(o, L)],
                    a_ys + stage_v[pl.ds(o + L, L)],
                    a_s + stage_v[pl.ds(o + 2 * L, L)])

        a_y, a_ys, a_s = lax.fori_loop(0, NT, comb, (zf, zf, zf))
        tot_y = jnp.full((L,), jnp.sum(a_y), jnp.float32)
        tot_ys = jnp.full((L,), jnp.sum(a_ys), jnp.float32)
        cnt1 = jnp.full((L,), jnp.sum(a_s), jnp.float32)
        mean1 = tot_ys / cnt1
        mean0 = (tot_y - tot_ys) / (jnp.float32(N) - cnt1)
        res_v[...] = jnp.abs(mean0 - mean1)
        pltpu.sync_copy(res_v, out_hbm)


def kernel(y_pred, s):
    y_flat = jnp.transpose(y_pred.reshape(N // 128, 128, 2), (0, 2, 1)).reshape(-1)
    out = _diffdp(y_flat, s.astype(jnp.int32))
    return out[0]


# speedup vs baseline: 1.0891x; 1.0891x over previous
import functools

import jax
import jax.numpy as jnp
from jax import lax
from jax.experimental import pallas as pl
from jax.experimental.pallas import tpu as pltpu
from jax.experimental.pallas import tpu_sc as plsc

N = 16384
L = 16
NT = 16
ROWS = N // NT
CHUNKS = ROWS // L

_mesh = plsc.VectorSubcoreMesh(
    core_axis_name="c", subcore_axis_name="s", num_cores=1)


@functools.partial(
    pl.kernel,
    mesh=_mesh,
    out_type=jax.ShapeDtypeStruct((L,), jnp.float32),
    compiler_params=pltpu.CompilerParams(
        needs_layout_passes=False,
        disable_bounds_checks=True,
        disable_semaphore_checks=True,
        skip_device_barrier=True,
    ),
    scratch_types=[
        pltpu.VMEM((2 * ROWS,), jnp.float32),
        pltpu.VMEM((ROWS,), jnp.int32),
        pltpu.VMEM((3 * L,), jnp.float32),
        pltpu.VMEM((NT * 3 * L,), jnp.float32),
        pltpu.VMEM_SHARED((NT * 3 * L,), jnp.float32),
        pltpu.VMEM((L,), jnp.float32),
        pltpu.SemaphoreType.DMA,
        pltpu.SemaphoreType.DMA,
    ],
)
def _diffdp(y_hbm, s_hbm, out_hbm, y_v, s_v, part_v, stage_v, shared, res_v,
            sem_y, sem_s):
    tid = lax.axis_index("s")
    base = tid * ROWS
    y_cp = pltpu.async_copy(y_hbm.at[pl.ds(2 * base, 2 * ROWS)], y_v, sem_y)
    s_cp = pltpu.async_copy(s_hbm.at[pl.ds(base, ROWS)], s_v, sem_s)
    s_cp.wait()
    y_cp.wait()

    zf = jnp.zeros((L,), jnp.float32)

    @plsc.parallel_loop(0, CHUNKS, carry=(zf, zf, zf), unroll=4)
    def acc(i, carry):
        acc_y, acc_ys, acc_s = carry
        off = L * i + 128 * (i // 8) + 128
        yv = y_v[pl.ds(off, L)]
        sv = s_v[pl.ds(i * L, L)].astype(jnp.float32)
        return acc_y + yv, acc_ys + yv * sv, acc_s + sv

    acc_y, acc_ys, acc_s = acc
    part_v[pl.ds(0, L)] = acc_y
    part_v[pl.ds(L, L)] = acc_ys
    part_v[pl.ds(2 * L, L)] = acc_s
    pltpu.sync_copy(part_v, shared.at[pl.ds(tid * 3 * L, 3 * L)])
    plsc.subcore_barrier()

    @pl.when(tid == 0)
    def _():
        pltpu.sync_copy(shared, stage_v)

        def comb(k, carry):
            a_y, a_ys, a_s = carry
            o = k * 3 * L
            return (a_y + stage_v[pl.ds(o, L)],
                    a_ys + stage_v[pl.ds(o + L, L)],
                    a_s + stage_v[pl.ds(o + 2 * L, L)])

        a_y, a_ys, a_s = lax.fori_loop(0, NT, comb, (zf, zf, zf))
        tot_y = jnp.full((L,), jnp.sum(a_y), jnp.float32)
        tot_ys = jnp.full((L,), jnp.sum(a_ys), jnp.float32)
        cnt1 = jnp.full((L,), jnp.sum(a_s), jnp.float32)
        mean1 = tot_ys / cnt1
        mean0 = (tot_y - tot_ys) / (jnp.float32(N) - cnt1)
        res_v[...] = jnp.abs(mean0 - mean1)
        pltpu.sync_copy(res_v, out_hbm)


def kernel(y_pred, s):
    y_flat = jnp.transpose(y_pred.reshape(N // 128, 128, 2), (0, 2, 1)).reshape(-1)
    out = _diffdp(y_flat, s.astype(jnp.int32))
    return out[0]
